# baseline (speedup 1.0000x reference)
import functools

import jax
import jax.numpy as jnp
from jax import lax
from jax.experimental import pallas as pl
from jax.experimental.pallas import tpu as pltpu
from jax.experimental.pallas import tpu_sc as plsc

N = 10000
E = 320000
IN_DIM = 128
HID = 256
EMB = 64

NC = 2
NS = 16
C = 128
CH = 160
EPS = CH * C
EPAD = NS * EPS
NP = 10240
HR = NP // C
RPS = NP // NS
CZ = 128
NWB = RPS // CZ
KB = 16
NB = CH // KB

_f32 = jnp.float32


def _make_sc_l1():
  mesh = plsc.VectorSubcoreMesh(core_axis_name="c", subcore_axis_name="s")

  out_type = [jax.ShapeDtypeStruct((NP, IN_DIM), _f32),
              jax.ShapeDtypeStruct((NP, IN_DIM), _f32)]

  scratch = [
      pltpu.VMEM((KB, C), jnp.int32),
      pltpu.VMEM((KB, C), jnp.int32),
      pltpu.VMEM((KB, C), jnp.int32),
      pltpu.VMEM((KB, C), jnp.int32),
      pltpu.VMEM((C, IN_DIM), _f32),
      pltpu.VMEM((C, IN_DIM), _f32),
      pltpu.SemaphoreType.DMA,
      pltpu.SemaphoreType.DMA,
      pltpu.SemaphoreType.DMA,
      pltpu.VMEM_SHARED((NP, IN_DIM), _f32),
  ]

  @functools.partial(pl.kernel, out_type=out_type, mesh=mesh,
                     scratch_types=scratch)
  def sc_kernel(xu, xi, su, du, si, di,
                agg_u, agg_i,
                srcA, dstA, srcB, dstB, rowb0, rowb1, gsem0, gsem1, isem,
                acc):
    rowb = (rowb0, rowb1)
    gsem = (gsem0, gsem1)
    slots = ((srcA, dstA), (srcB, dstB))
    c = lax.axis_index("c")
    s = lax.axis_index("s")
    base = s * RPS

    zeros16 = jnp.zeros((16,), _f32)

    def fill_rowb0(i, carry):
      rowb0[i // 8, pl.ds((i % 8) * 16, 16)] = zeros16
      return carry

    lax.fori_loop(0, C * IN_DIM // 16, fill_rowb0, 0)

    for k in range(NWB):
      pltpu.sync_copy(rowb0, acc.at[pl.ds(base + k * CZ, CZ)])

    plsc.subcore_barrier()

    def run(x_hbm, src_hbm, dst_hbm):
      pltpu.sync_copy(src_hbm.at[s, pl.ds(0, KB)], srcA)
      pltpu.sync_copy(dst_hbm.at[s, pl.ds(0, KB)], dstA)
      pltpu.async_copy(x_hbm.at[srcA.at[0]], rowb[0], gsem[0])

      def block(kb, carry):
        def process(cur, other):
          csrc, cdst = cur
          osrc, odst = other
          last_blk = kb + 1 >= NB

          @pl.when(jnp.logical_not(last_blk))
          def _():
            pltpu.async_copy(src_hbm.at[s, pl.ds((kb + 1) * KB, KB)],
                             osrc, isem)
            pltpu.async_copy(dst_hbm.at[s, pl.ds((kb + 1) * KB, KB)],
                             odst, isem)

          for u in range(KB):
            if u == KB - 1:
              @pl.when(jnp.logical_not(last_blk))
              def _():
                pltpu.make_async_copy(
                    src_hbm.at[s, pl.ds((kb + 1) * KB, KB)], osrc,
                    isem).wait()
                pltpu.make_async_copy(
                    dst_hbm.at[s, pl.ds((kb + 1) * KB, KB)], odst,
                    isem).wait()

              @pl.when(jnp.logical_not(last_blk))
              def _():
                pltpu.async_copy(x_hbm.at[osrc.at[0]],
                                 rowb[(u + 1) % 2], gsem[(u + 1) % 2])
            else:
              pltpu.async_copy(x_hbm.at[csrc.at[u + 1]],
                               rowb[(u + 1) % 2], gsem[(u + 1) % 2])

            pltpu.make_async_copy(x_hbm.at[csrc.at[u]], rowb[u % 2],
                                  gsem[u % 2]).wait()
            pltpu.sync_copy(rowb[u % 2], acc.at[cdst.at[u]], add=True)

        @pl.when(kb % 2 == 0)
        def _():
          process(slots[0], slots[1])

        @pl.when(kb % 2 == 1)
        def _():
          process(slots[1], slots[0])

        return carry

      lax.fori_loop(0, NB, block, 0)

    @pl.when(c == 0)
    def _():
      run(xu, su, du)

    @pl.when(c == 1)
    def _():
      run(xi, si, di)

    plsc.subcore_barrier()

    def writeback(agg_hbm):
      for k in range(NWB):
        off = base + k * CZ
        pltpu.sync_copy(acc.at[pl.ds(off, CZ)], rowb0)
        pltpu.sync_copy(rowb0, agg_hbm.at[pl.ds(off, CZ)])

    @pl.when(c == 0)
    def _():
      writeback(agg_u)

    @pl.when(c == 1)
    def _():
      writeback(agg_i)

  return sc_kernel


def _make_sc_deg():
  mesh = plsc.VectorSubcoreMesh(core_axis_name="c", subcore_axis_name="s")

  out_type = [jax.ShapeDtypeStruct((NS, HR, C), _f32),
              jax.ShapeDtypeStruct((NS, HR, C), _f32),
              jax.ShapeDtypeStruct((HR, C), _f32),
              jax.ShapeDtypeStruct((HR, C), _f32)]

  scratch = [
      pltpu.VMEM((CH, C), jnp.int32),
      pltpu.VMEM((HR, C), _f32),
      pltpu.VMEM((8, C), _f32),
      pltpu.VMEM((8, C), _f32),
  ]

  @functools.partial(pl.kernel, out_type=out_type, mesh=mesh,
                     scratch_types=scratch,
                     compiler_params=pltpu.CompilerParams(
                         needs_layout_passes=False))
  def sc_kernel(du, di, dp_u, dp_i, deg_u, deg_i, dstv, hist, racc, rbuf):
    c = lax.axis_index("c")
    s = lax.axis_index("s")

    zeros16 = jnp.zeros((16,), _f32)
    ones16 = jnp.ones((16,), _f32)

    def fill_hist(i, carry):
      hist[i // 8, pl.ds((i % 8) * 16, 16)] = zeros16
      return carry

    lax.fori_loop(0, HR * C // 16, fill_hist, 0)

    @pl.when(c == 0)
    def _():
      pltpu.sync_copy(du.at[s], dstv)

    @pl.when(c == 1)
    def _():
      pltpu.sync_copy(di.at[s], dstv)

    def count(j, carry):
      for u in range(C // 16):
        idx = dstv[j, pl.ds(u * 16, 16)]
        plsc.addupdate_scatter(
            hist, [lax.shift_right_logical(idx, 7),
                   lax.bitwise_and(idx, 127)], ones16)
      return carry

    lax.fori_loop(0, CH, count, 0)

    def publish(dp_hbm):
      pltpu.sync_copy(hist, dp_hbm.at[s])

    @pl.when(c == 0)
    def _():
      publish(dp_u)

    @pl.when(c == 1)
    def _():
      publish(dp_i)

    plsc.subcore_barrier()

    @pl.when(s < NS - 6)
    def _():
      def fill_racc(i, carry):
        racc[i // 8, pl.ds((i % 8) * 16, 16)] = zeros16
        return carry

      lax.fori_loop(0, 8 * C // 16, fill_racc, 0)

      def reduce_from(dp_hbm):
        for t in range(NS):
          pltpu.sync_copy(dp_hbm.at[t, pl.ds(8 * s, 8)], rbuf)

          def add_rows(i, carry):
            sl = pl.ds((i % 8) * 16, 16)
            racc[i // 8, sl] = racc[i // 8, sl] + rbuf[i // 8, sl]
            return carry

          lax.fori_loop(0, 8 * C // 16, add_rows, 0)

      @pl.when(c == 0)
      def _():
        reduce_from(dp_u)
        pltpu.sync_copy(racc, deg_u.at[pl.ds(8 * s, 8)])

      @pl.when(c == 1)
      def _():
        reduce_from(dp_i)
        pltpu.sync_copy(racc, deg_i.at[pl.ds(8 * s, 8)])

  return sc_kernel


_sc_segsum = _make_sc_l1()
_sc_deg = _make_sc_deg()


_BT = 1024
_GT = NP // _BT
_HB = _BT // C


def _dot_t(a, w):
  return lax.dot_general(a, w, (((1,), (1,)), ((), ())),
                         preferred_element_type=_f32)


def _tc_mid_body(agg_u, deg_u, xu, agg_i, deg_i, xi,
                 uWl1, ubl1, uWr1, uWl2, ubl2, uWr2,
                 iWl1, ibl1, iWr1, iWl2, ibl2, iWr2,
                 pcat, r_u, r_i):
  def enc(agg, deg, x, Wl1, bl1, Wr1, Wl2, bl2, Wr2, r_out):
    d = jnp.maximum(deg[...], 1.0)
    mean = agg[...] / d
    h = jnp.maximum(_dot_t(mean, Wl1[...]) + bl1[...] + _dot_t(x[...], Wr1[...]),
                    0.0)
    r_out[...] = _dot_t(h, Wr2[...]) + bl2[...]
    return _dot_t(h, Wl2[...])

  p_u = enc(agg_u, deg_u, xu, uWl1, ubl1, uWr1, uWl2, ubl2, uWr2, r_u)
  p_i = enc(agg_i, deg_i, xi, iWl1, ibl1, iWr1, iWl2, ibl2, iWr2, r_i)
  pcat[...] = jnp.concatenate([p_u, p_i], axis=1)


def _row_spec(d):
  return pl.BlockSpec((_BT, d), lambda i: (i, 0))


def _full_spec(shape):
  nd = len(shape)
  return pl.BlockSpec(shape, lambda i: (0,) * nd)


def _tc_mid(agg_u, deg_u, xu, agg_i, deg_i, xi, wu, wi):
  w_specs = [_full_spec(w.shape) for w in (wu + wi)]
  return pl.pallas_call(
      _tc_mid_body,
      grid=(_GT,),
      in_specs=[_row_spec(IN_DIM), _row_spec(1), _row_spec(IN_DIM),
                _row_spec(IN_DIM), _row_spec(1), _row_spec(IN_DIM)] + w_specs,
      out_specs=[_row_spec(IN_DIM), _row_spec(EMB), _row_spec(EMB)],
      out_shape=[jax.ShapeDtypeStruct((NP, IN_DIM), _f32),
                 jax.ShapeDtypeStruct((NP, EMB), _f32),
                 jax.ShapeDtypeStruct((NP, EMB), _f32)],
  )(agg_u, deg_u, xu, agg_i, deg_i, xi, *wu, *wi)


def _tc_head_body(a2u, deg_u, ru, a2i, deg_i, ri, sW, sb, out):
  eu = a2u[...][:, :EMB] / jnp.maximum(deg_u[...], 1.0) + ru[...]
  ei = a2i[...][:, EMB:] / jnp.maximum(deg_i[...], 1.0) + ri[...]
  w = sW[...]
  z = _dot_t(eu, w[:, :EMB]) + _dot_t(ei, w[:, EMB:]) + sb[...]
  out[...] = 1.0 / (1.0 + jnp.exp(-z))


def _tc_head(a2u, deg_u, ru, a2i, deg_i, ri, sW, sb):
  return pl.pallas_call(
      _tc_head_body,
      grid=(_GT,),
      in_specs=[_row_spec(IN_DIM), _row_spec(1), _row_spec(EMB),
                _row_spec(IN_DIM), _row_spec(1), _row_spec(EMB),
                _full_spec((1, 2 * EMB)), _full_spec((1, 1))],
      out_specs=_row_spec(1),
      out_shape=jax.ShapeDtypeStruct((NP, 1), _f32),
  )(a2u, deg_u, ru, a2i, deg_i, ri, sW, sb)


def kernel(user_x, item_x, user_edge_index, item_edge_index,
           u_Wl1, u_bl1, u_Wr1, u_Wl2, u_bl2, u_Wr2,
           i_Wl1, i_bl1, i_Wr1, i_Wl2, i_bl2, i_Wr2,
           s_W, s_b):
  npad = EPAD - E

  def edges(ei):
    src = jnp.concatenate(
        [ei[0].astype(jnp.int32), jnp.zeros((npad,), jnp.int32)])
    dst = jnp.concatenate(
        [ei[1].astype(jnp.int32), jnp.full((npad,), NP - 1, jnp.int32)])
    return src.reshape(NS, CH, C), dst.reshape(NS, CH, C)

  su, du = edges(user_edge_index)
  si, di = edges(item_edge_index)

  xu = jnp.pad(user_x, ((0, NP - N), (0, 0)))
  xi = jnp.pad(item_x, ((0, NP - N), (0, 0)))

  _, _, deg80_u, deg80_i = _sc_deg(du, di)
  deg_u = deg80_u.reshape(NP, 1)
  deg_i = deg80_i.reshape(NP, 1)

  agg_u, agg_i = _sc_segsum(xu, xi, su, du, si, di)

  wu = (u_Wl1, u_bl1.reshape(1, HID), u_Wr1,
        u_Wl2, u_bl2.reshape(1, EMB), u_Wr2)
  wi = (i_Wl1, i_bl1.reshape(1, HID), i_Wr1,
        i_Wl2, i_bl2.reshape(1, EMB), i_Wr2)
  pcat, r_u, r_i = _tc_mid(agg_u, deg_u, xu, agg_i, deg_i, xi, wu, wi)

  agg2_u, agg2_i = _sc_segsum(pcat, pcat, su, du, si, di)

  out = _tc_head(agg2_u, deg_u, r_u, agg2_i, deg_i, r_i,
                 s_W, s_b.reshape(1, 1))
  return out[:N]


# speedup vs baseline: 1.0233x; 1.0233x over previous
import functools

import jax
import jax.numpy as jnp
from jax import lax
from jax.experimental import pallas as pl
from jax.experimental.pallas import tpu as pltpu
from jax.experimental.pallas import tpu_sc as plsc

N = 10000
E = 320000
IN_DIM = 128
HID = 256
EMB = 64

NC = 2
NS = 16
C = 128
CH = 160
EPS = CH * C
EPAD = NS * EPS
NP = 10240
HR = NP // C
RPS = NP // NS
CZ = 128
NWB = RPS // CZ
KB = 16
NB = CH // KB

_f32 = jnp.float32


def _make_sc_l1():
  mesh = plsc.VectorSubcoreMesh(core_axis_name="c", subcore_axis_name="s")

  out_type = [jax.ShapeDtypeStruct((NP, IN_DIM), _f32),
              jax.ShapeDtypeStruct((NP, IN_DIM), _f32)]

  scratch = [
      pltpu.VMEM((KB, C), jnp.int32),
      pltpu.VMEM((KB, C), jnp.int32),
      pltpu.VMEM((KB, C), jnp.int32),
      pltpu.VMEM((KB, C), jnp.int32),
      pltpu.VMEM((C, IN_DIM), _f32),
      pltpu.VMEM((C, IN_DIM), _f32),
      pltpu.SemaphoreType.DMA,
      pltpu.SemaphoreType.DMA,
      pltpu.SemaphoreType.DMA,
      pltpu.VMEM_SHARED((NP, IN_DIM), _f32),
  ]

  @functools.partial(pl.kernel, out_type=out_type, mesh=mesh,
                     scratch_types=scratch,
                     compiler_params=pltpu.CompilerParams(
                         use_tc_tiling_on_sc=False))
  def sc_kernel(xu, xi, su, du, si, di,
                agg_u, agg_i,
                srcA, dstA, srcB, dstB, rowb0, rowb1, gsem0, gsem1, isem,
                acc):
    rowb = (rowb0, rowb1)
    gsem = (gsem0, gsem1)
    slots = ((srcA, dstA), (srcB, dstB))
    c = lax.axis_index("c")
    s = lax.axis_index("s")
    base = s * RPS

    zeros16 = jnp.zeros((16,), _f32)

    def fill_rowb0(i, carry):
      rowb0[i // 8, pl.ds((i % 8) * 16, 16)] = zeros16
      return carry

    lax.fori_loop(0, C * IN_DIM // 16, fill_rowb0, 0)

    for k in range(NWB):
      pltpu.sync_copy(rowb0, acc.at[pl.ds(base + k * CZ, CZ)])

    plsc.subcore_barrier()

    def run(x_hbm, src_hbm, dst_hbm):
      pltpu.sync_copy(src_hbm.at[s, pl.ds(0, KB)], srcA)
      pltpu.sync_copy(dst_hbm.at[s, pl.ds(0, KB)], dstA)
      pltpu.async_copy(x_hbm.at[srcA.at[0]], rowb[0], gsem[0])

      def block(kb, carry):
        def process(cur, other):
          csrc, cdst = cur
          osrc, odst = other
          last_blk = kb + 1 >= NB

          @pl.when(jnp.logical_not(last_blk))
          def _():
            pltpu.async_copy(src_hbm.at[s, pl.ds((kb + 1) * KB, KB)],
                             osrc, isem)
            pltpu.async_copy(dst_hbm.at[s, pl.ds((kb + 1) * KB, KB)],
                             odst, isem)

          for u in range(KB):
            if u == KB - 1:
              @pl.when(jnp.logical_not(last_blk))
              def _():
                pltpu.make_async_copy(
                    src_hbm.at[s, pl.ds((kb + 1) * KB, KB)], osrc,
                    isem).wait()
                pltpu.make_async_copy(
                    dst_hbm.at[s, pl.ds((kb + 1) * KB, KB)], odst,
                    isem).wait()

              @pl.when(jnp.logical_not(last_blk))
              def _():
                pltpu.async_copy(x_hbm.at[osrc.at[0]],
                                 rowb[(u + 1) % 2], gsem[(u + 1) % 2])
            else:
              pltpu.async_copy(x_hbm.at[csrc.at[u + 1]],
                               rowb[(u + 1) % 2], gsem[(u + 1) % 2])

            pltpu.make_async_copy(x_hbm.at[csrc.at[u]], rowb[u % 2],
                                  gsem[u % 2]).wait()
            pltpu.sync_copy(rowb[u % 2], acc.at[cdst.at[u]], add=True)

        @pl.when(kb % 2 == 0)
        def _():
          process(slots[0], slots[1])

        @pl.when(kb % 2 == 1)
        def _():
          process(slots[1], slots[0])

        return carry

      lax.fori_loop(0, NB, block, 0)

    @pl.when(c == 0)
    def _():
      run(xu, su, du)

    @pl.when(c == 1)
    def _():
      run(xi, si, di)

    plsc.subcore_barrier()

    def writeback(agg_hbm):
      for k in range(NWB):
        off = base + k * CZ
        pltpu.sync_copy(acc.at[pl.ds(off, CZ)], rowb0)
        pltpu.sync_copy(rowb0, agg_hbm.at[pl.ds(off, CZ)])

    @pl.when(c == 0)
    def _():
      writeback(agg_u)

    @pl.when(c == 1)
    def _():
      writeback(agg_i)

  return sc_kernel


def _make_sc_deg():
  mesh = plsc.VectorSubcoreMesh(core_axis_name="c", subcore_axis_name="s")

  out_type = [jax.ShapeDtypeStruct((NS, HR, C), _f32),
              jax.ShapeDtypeStruct((NS, HR, C), _f32),
              jax.ShapeDtypeStruct((HR, C), _f32),
              jax.ShapeDtypeStruct((HR, C), _f32)]

  scratch = [
      pltpu.VMEM((CH, C), jnp.int32),
      pltpu.VMEM((HR, C), _f32),
      pltpu.VMEM((8, C), _f32),
      pltpu.VMEM((8, C), _f32),
  ]

  @functools.partial(pl.kernel, out_type=out_type, mesh=mesh,
                     scratch_types=scratch,
                     compiler_params=pltpu.CompilerParams(
                         needs_layout_passes=False))
  def sc_kernel(du, di, dp_u, dp_i, deg_u, deg_i, dstv, hist, racc, rbuf):
    c = lax.axis_index("c")
    s = lax.axis_index("s")

    zeros16 = jnp.zeros((16,), _f32)
    ones16 = jnp.ones((16,), _f32)

    def fill_hist(i, carry):
      hist[i // 8, pl.ds((i % 8) * 16, 16)] = zeros16
      return carry

    lax.fori_loop(0, HR * C // 16, fill_hist, 0)

    @pl.when(c == 0)
    def _():
      pltpu.sync_copy(du.at[s], dstv)

    @pl.when(c == 1)
    def _():
      pltpu.sync_copy(di.at[s], dstv)

    def count(j, carry):
      for u in range(C // 16):
        idx = dstv[j, pl.ds(u * 16, 16)]
        plsc.addupdate_scatter(
            hist, [lax.shift_right_logical(idx, 7),
                   lax.bitwise_and(idx, 127)], ones16)
      return carry

    lax.fori_loop(0, CH, count, 0)

    def publish(dp_hbm):
      pltpu.sync_copy(hist, dp_hbm.at[s])

    @pl.when(c == 0)
    def _():
      publish(dp_u)

    @pl.when(c == 1)
    def _():
      publish(dp_i)

    plsc.subcore_barrier()

    @pl.when(s < NS - 6)
    def _():
      def fill_racc(i, carry):
        racc[i // 8, pl.ds((i % 8) * 16, 16)] = zeros16
        return carry

      lax.fori_loop(0, 8 * C // 16, fill_racc, 0)

      def reduce_from(dp_hbm):
        for t in range(NS):
          pltpu.sync_copy(dp_hbm.at[t, pl.ds(8 * s, 8)], rbuf)

          def add_rows(i, carry):
            sl = pl.ds((i % 8) * 16, 16)
            racc[i // 8, sl] = racc[i // 8, sl] + rbuf[i // 8, sl]
            return carry

          lax.fori_loop(0, 8 * C // 16, add_rows, 0)

      @pl.when(c == 0)
      def _():
        reduce_from(dp_u)
        pltpu.sync_copy(racc, deg_u.at[pl.ds(8 * s, 8)])

      @pl.when(c == 1)
      def _():
        reduce_from(dp_i)
        pltpu.sync_copy(racc, deg_i.at[pl.ds(8 * s, 8)])

  return sc_kernel


_sc_segsum = _make_sc_l1()
_sc_deg = _make_sc_deg()


_BT = 1024
_GT = NP // _BT
_HB = _BT // C


def _dot_t(a, w):
  return lax.dot_general(a, w, (((1,), (1,)), ((), ())),
                         preferred_element_type=_f32)


def _tc_mid_body(agg_u, deg_u, xu, agg_i, deg_i, xi,
                 uWl1, ubl1, uWr1, uWl2, ubl2, uWr2,
                 iWl1, ibl1, iWr1, iWl2, ibl2, iWr2,
                 pcat, r_u, r_i):
  def enc(agg, deg, x, Wl1, bl1, Wr1, Wl2, bl2, Wr2, r_out):
    d = jnp.maximum(deg[...], 1.0)
    mean = agg[...] / d
    h = jnp.maximum(_dot_t(mean, Wl1[...]) + bl1[...] + _dot_t(x[...], Wr1[...]),
                    0.0)
    r_out[...] = _dot_t(h, Wr2[...]) + bl2[...]
    return _dot_t(h, Wl2[...])

  p_u = enc(agg_u, deg_u, xu, uWl1, ubl1, uWr1, uWl2, ubl2, uWr2, r_u)
  p_i = enc(agg_i, deg_i, xi, iWl1, ibl1, iWr1, iWl2, ibl2, iWr2, r_i)
  pcat[...] = jnp.concatenate([p_u, p_i], axis=1)


def _row_spec(d):
  return pl.BlockSpec((_BT, d), lambda i: (i, 0))


def _full_spec(shape):
  nd = len(shape)
  return pl.BlockSpec(shape, lambda i: (0,) * nd)


def _tc_mid(agg_u, deg_u, xu, agg_i, deg_i, xi, wu, wi):
  w_specs = [_full_spec(w.shape) for w in (wu + wi)]
  return pl.pallas_call(
      _tc_mid_body,
      grid=(_GT,),
      in_specs=[_row_spec(IN_DIM), _row_spec(1), _row_spec(IN_DIM),
                _row_spec(IN_DIM), _row_spec(1), _row_spec(IN_DIM)] + w_specs,
      out_specs=[_row_spec(IN_DIM), _row_spec(EMB), _row_spec(EMB)],
      out_shape=[jax.ShapeDtypeStruct((NP, IN_DIM), _f32),
                 jax.ShapeDtypeStruct((NP, EMB), _f32),
                 jax.ShapeDtypeStruct((NP, EMB), _f32)],
  )(agg_u, deg_u, xu, agg_i, deg_i, xi, *wu, *wi)


def _tc_head_body(a2u, deg_u, ru, a2i, deg_i, ri, sW, sb, out):
  eu = a2u[...][:, :EMB] / jnp.maximum(deg_u[...], 1.0) + ru[...]
  ei = a2i[...][:, EMB:] / jnp.maximum(deg_i[...], 1.0) + ri[...]
  w = sW[...]
  z = _dot_t(eu, w[:, :EMB]) + _dot_t(ei, w[:, EMB:]) + sb[...]
  out[...] = 1.0 / (1.0 + jnp.exp(-z))


def _tc_head(a2u, deg_u, ru, a2i, deg_i, ri, sW, sb):
  return pl.pallas_call(
      _tc_head_body,
      grid=(_GT,),
      in_specs=[_row_spec(IN_DIM), _row_spec(1), _row_spec(EMB),
                _row_spec(IN_DIM), _row_spec(1), _row_spec(EMB),
                _full_spec((1, 2 * EMB)), _full_spec((1, 1))],
      out_specs=_row_spec(1),
      out_shape=jax.ShapeDtypeStruct((NP, 1), _f32),
  )(a2u, deg_u, ru, a2i, deg_i, ri, sW, sb)


def kernel(user_x, item_x, user_edge_index, item_edge_index,
           u_Wl1, u_bl1, u_Wr1, u_Wl2, u_bl2, u_Wr2,
           i_Wl1, i_bl1, i_Wr1, i_Wl2, i_bl2, i_Wr2,
           s_W, s_b):
  npad = EPAD - E

  def edges(ei):
    src = jnp.concatenate(
        [ei[0].astype(jnp.int32), jnp.zeros((npad,), jnp.int32)])
    dst = jnp.concatenate(
        [ei[1].astype(jnp.int32), jnp.full((npad,), NP - 1, jnp.int32)])
    return src.reshape(NS, CH, C), dst.reshape(NS, CH, C)

  su, du = edges(user_edge_index)
  si, di = edges(item_edge_index)

  xu = jnp.pad(user_x, ((0, NP - N), (0, 0)))
  xi = jnp.pad(item_x, ((0, NP - N), (0, 0)))

  _, _, deg80_u, deg80_i = _sc_deg(du, di)
  deg_u = deg80_u.reshape(NP, 1)
  deg_i = deg80_i.reshape(NP, 1)

  agg_u, agg_i = _sc_segsum(xu, xi, su, du, si, di)

  wu = (u_Wl1, u_bl1.reshape(1, HID), u_Wr1,
        u_Wl2, u_bl2.reshape(1, EMB), u_Wr2)
  wi = (i_Wl1, i_bl1.reshape(1, HID), i_Wr1,
        i_Wl2, i_bl2.reshape(1, EMB), i_Wr2)
  pcat, r_u, r_i = _tc_mid(agg_u, deg_u, xu, agg_i, deg_i, xi, wu, wi)

  agg2_u, agg2_i = _sc_segsum(pcat, pcat, su, du, si, di)

  out = _tc_head(agg2_u, deg_u, r_u, agg2_i, deg_i, r_i,
                 s_W, s_b.reshape(1, 1))
  return out[:N]
